# baseline (speedup 1.0000x reference)
import jax
import jax.numpy as jnp
from jax.experimental import pallas as pl
from jax.experimental.pallas import tpu as pltpu


def kernel(tokens, W_pos):
    B, P = tokens.shape
    D = W_pos.shape[1]
    CHUNKS = (512, 512, 1024, 1024, 1024)
    assert sum(CHUNKS) == P
    OFFS = [0]
    for c in CHUNKS:
        OFFS.append(OFFS[-1] + c)
    NC = len(CHUNKS)

    def body(w_hbm, o_hbm, buf, in_sem, out_sem):
        def in_copy(i):
            return pltpu.make_async_copy(
                w_hbm.at[pl.ds(OFFS[i], CHUNKS[i]), :],
                buf.at[pl.ds(OFFS[i], CHUNKS[i]), :],
                in_sem.at[i])

        def out_copy(i, b):
            return pltpu.make_async_copy(
                buf.at[pl.ds(OFFS[i], CHUNKS[i]), :],
                o_hbm.at[b, pl.ds(OFFS[i], CHUNKS[i]), :],
                out_sem.at[i, b])

        for i in range(NC):
            in_copy(i).start()
        for i in range(NC):
            in_copy(i).wait()
            for b in range(B):
                out_copy(i, b).start()
        for i in range(NC):
            for b in range(B):
                out_copy(i, b).wait()

    out = pl.pallas_call(
        body,
        in_specs=[pl.BlockSpec(memory_space=pl.ANY)],
        out_specs=pl.BlockSpec(memory_space=pl.ANY),
        out_shape=jax.ShapeDtypeStruct((B, P, D), W_pos.dtype),
        scratch_shapes=[
            pltpu.VMEM((P, D), W_pos.dtype),
            pltpu.SemaphoreType.DMA((NC,)),
            pltpu.SemaphoreType.DMA((NC, B)),
        ],
    )(W_pos)
    return out


# speedup vs baseline: 1.0069x; 1.0069x over previous
import jax
import jax.numpy as jnp
from jax.experimental import pallas as pl
from jax.experimental.pallas import tpu as pltpu


def kernel(tokens, W_pos):
    B, P = tokens.shape
    D = W_pos.shape[1]
    CHUNK = 1024
    NC = P // CHUNK

    def body(w_hbm, o_hbm, buf, in_sem, out_sem):
        def in_copy(i):
            return pltpu.make_async_copy(
                w_hbm.at[pl.ds(i * CHUNK, CHUNK), :],
                buf.at[pl.ds(i * CHUNK, CHUNK), :],
                in_sem.at[i])

        def out_copy(i, b):
            return pltpu.make_async_copy(
                buf.at[pl.ds(i * CHUNK, CHUNK), :],
                o_hbm.at[b, pl.ds(i * CHUNK, CHUNK), :],
                out_sem.at[i, b])

        for i in range(NC):
            in_copy(i).start()
        for i in range(NC):
            in_copy(i).wait()
            for b in range(B):
                out_copy(i, b).start()
        for i in range(NC):
            for b in range(B):
                out_copy(i, b).wait()

    out = pl.pallas_call(
        body,
        in_specs=[pl.BlockSpec(memory_space=pl.ANY)],
        out_specs=pl.BlockSpec(memory_space=pl.ANY),
        out_shape=jax.ShapeDtypeStruct((B, P, D), W_pos.dtype),
        scratch_shapes=[
            pltpu.VMEM((P, D), W_pos.dtype),
            pltpu.SemaphoreType.DMA((NC,)),
            pltpu.SemaphoreType.DMA((NC, B)),
        ],
    )(W_pos)
    return out
